# Initial kernel scaffold; baseline (speedup 1.0000x reference)
import jax
import jax.numpy as jnp
from jax.experimental import pallas as pl


def kernel(x, edge_index, W1, W2):
    raise NotImplementedError("write your pallas kernel here")


# speedup vs baseline: 8.3905x; 8.3905x over previous
import functools

import jax
import jax.numpy as jnp
from jax import lax
from jax.experimental import pallas as pl
from jax.experimental.pallas import tpu as pltpu
from jax.experimental.pallas import tpu_sc as plsc

NUM_CORES = 2
NUM_SUBCORES = 16
CHUNK = 128


def _edge_scatter(rows_tbl, srcp, dstp, n_pad, feat, const_rows):
    e_pad = dstp.shape[0]
    cpt = e_pad // (NUM_CORES * NUM_SUBCORES * CHUNK)
    rpt = n_pad // NUM_SUBCORES
    mesh = plsc.VectorSubcoreMesh(core_axis_name="c", subcore_axis_name="s")
    zeros = jnp.zeros((rpt, feat), jnp.float32)

    scratch = [
        pltpu.VMEM((CHUNK,), jnp.int32),
        pltpu.VMEM((CHUNK, feat), jnp.float32),
        pltpu.VMEM_SHARED((n_pad, feat), jnp.float32),
        pltpu.SemaphoreType.DMA,
    ]
    if not const_rows:
        scratch.insert(0, pltpu.VMEM((CHUNK,), jnp.int32))

    @functools.partial(
        pl.kernel,
        mesh=mesh,
        out_type=jax.ShapeDtypeStruct((NUM_CORES * n_pad, feat), jnp.float32),
        scratch_types=scratch,
        compiler_params=pltpu.CompilerParams(use_tc_tiling_on_sc=False),
    )
    def k(rows_hbm, src_hbm, dst_hbm, zeros_hbm, out_hbm, *refs):
        if const_rows:
            dstv, rows, acc, sem = refs
            srcv = None
        else:
            srcv, dstv, rows, acc, sem = refs
        c = lax.axis_index("c")
        s = lax.axis_index("s")
        pltpu.sync_copy(zeros_hbm, acc.at[pl.ds(s * rpt, rpt)])
        if const_rows:
            pltpu.sync_copy(rows_hbm, rows)
        plsc.subcore_barrier()

        wid = c * NUM_SUBCORES + s
        base_chunk = wid * cpt

        def body(i, carry):
            e0 = (base_chunk + i) * CHUNK
            pltpu.sync_copy(dst_hbm.at[pl.ds(e0, CHUNK)], dstv)
            if not const_rows:
                pltpu.sync_copy(src_hbm.at[pl.ds(e0, CHUNK)], srcv)
                pltpu.async_copy(rows_hbm.at[srcv], rows, sem).wait()
            pltpu.sync_copy(rows, acc.at[dstv], add=True)
            return carry

        lax.fori_loop(0, cpt, body, 0)
        plsc.subcore_barrier()
        pltpu.sync_copy(
            acc.at[pl.ds(s * rpt, rpt)],
            out_hbm.at[pl.ds(c * n_pad + s * rpt, rpt)],
        )

    return k(rows_tbl, srcp, dstp, zeros)


def _dinv(d0_ref, d1_ref):
    deg = d0_ref[:, :1] + d1_ref[:, :1] + 1.0
    return lax.rsqrt(jnp.maximum(deg, 1.0))


def _enc1(x, W1, d0, d1):
    n, dfe = x.shape
    hid = W1.shape[1]
    blk = 2000

    def body(x_ref, w_ref, d0_ref, d1_ref, s_ref):
        dinv = _dinv(d0_ref, d1_ref)
        s_ref[...] = jnp.dot(x_ref[...], w_ref[...],
                             preferred_element_type=jnp.float32) * dinv

    return pl.pallas_call(
        body,
        grid=(n // blk,),
        in_specs=[
            pl.BlockSpec((blk, dfe), lambda i: (i, 0)),
            pl.BlockSpec((dfe, hid), lambda i: (0, 0)),
            pl.BlockSpec((blk, d0.shape[1]), lambda i: (i, 0)),
            pl.BlockSpec((blk, d1.shape[1]), lambda i: (i, 0)),
        ],
        out_specs=pl.BlockSpec((blk, hid), lambda i: (i, 0)),
        out_shape=jax.ShapeDtypeStruct((n, hid), jnp.float32),
    )(x, W1, d0, d1)


def _enc2(p0, p1, s, W2, d0, d1):
    n, hid = s.shape
    code = W2.shape[1]
    blk = 2000

    def body(p0_ref, p1_ref, s_ref, w_ref, d0_ref, d1_ref, t_ref):
        dinv = _dinv(d0_ref, d1_ref)
        h = jnp.maximum((p0_ref[...] + p1_ref[...] + s_ref[...]) * dinv, 0.0)
        t_ref[...] = jnp.dot(h, w_ref[...],
                             preferred_element_type=jnp.float32) * dinv

    rspec = lambda f: pl.BlockSpec((blk, f), lambda i: (i, 0))
    return pl.pallas_call(
        body,
        grid=(n // blk,),
        in_specs=[rspec(hid), rspec(hid), rspec(hid),
                  pl.BlockSpec((hid, code), lambda i: (0, 0)),
                  rspec(d0.shape[1]), rspec(d1.shape[1])],
        out_specs=rspec(code),
        out_shape=jax.ShapeDtypeStruct((n, code), jnp.float32),
    )(p0, p1, s, W2, d0, d1)


def _form_z(q0, q1, t, d0, d1):
    n, code = t.shape
    blk = 2000

    def body(q0_ref, q1_ref, t_ref, d0_ref, d1_ref, z_ref):
        dinv = _dinv(d0_ref, d1_ref)
        z_ref[...] = (q0_ref[...] + q1_ref[...] + t_ref[...]) * dinv

    rspec = lambda f: pl.BlockSpec((blk, f), lambda i: (i, 0))
    return pl.pallas_call(
        body,
        grid=(n // blk,),
        in_specs=[rspec(code), rspec(code), rspec(code),
                  rspec(d0.shape[1]), rspec(d1.shape[1])],
        out_specs=rspec(code),
        out_shape=jax.ShapeDtypeStruct((n, code), jnp.float32),
    )(q0, q1, t, d0, d1)


def _decode(z):
    n, code = z.shape
    rb, cb = 512, 1024
    gi = (n + rb - 1) // rb
    gj = (n + cb - 1) // cb

    def body(zi_ref, zj_ref, o_ref):
        g = lax.dot_general(zi_ref[...], zj_ref[...],
                            (((1,), (1,)), ((), ())),
                            preferred_element_type=jnp.float32)
        o_ref[...] = jax.nn.sigmoid(g)

    return pl.pallas_call(
        body,
        grid=(gi, gj),
        in_specs=[
            pl.BlockSpec((rb, code), lambda i, j: (i, 0)),
            pl.BlockSpec((cb, code), lambda i, j: (j, 0)),
        ],
        out_specs=pl.BlockSpec((rb, cb), lambda i, j: (i, j)),
        out_shape=jax.ShapeDtypeStruct((n, n), jnp.float32),
    )(z, z)


def kernel(x, edge_index, W1, W2):
    n = x.shape[0]
    e = edge_index.shape[1]

    epc = NUM_CORES * NUM_SUBCORES * CHUNK
    e_pad = ((e + epc - 1) // epc) * epc
    src = edge_index[0]
    dst = edge_index[1]
    srcp = jnp.concatenate([src, jnp.zeros((e_pad - e,), jnp.int32)])
    dstp = jnp.concatenate([dst, jnp.full((e_pad - e,), n, jnp.int32)])

    n_pad = ((n + 1 + NUM_SUBCORES * 8 - 1) // (NUM_SUBCORES * 8)) * (NUM_SUBCORES * 8)

    degf = 16
    ones = jnp.ones((CHUNK, degf), jnp.float32)
    degp = _edge_scatter(ones, srcp, dstp, n_pad, degf, const_rows=True)
    d0 = degp[:n]
    d1 = degp[n_pad:n_pad + n]

    s1 = _enc1(x, W1, d0, d1)
    pp = _edge_scatter(s1, srcp, dstp, n_pad, s1.shape[1], const_rows=False)
    t = _enc2(pp[:n], pp[n_pad:n_pad + n], s1, W2, d0, d1)
    qq = _edge_scatter(t, srcp, dstp, n_pad, t.shape[1], const_rows=False)
    z = _form_z(qq[:n], qq[n_pad:n_pad + n], t, d0, d1)
    return _decode(z)


# speedup vs baseline: 9.8891x; 1.1786x over previous
import functools

import jax
import jax.numpy as jnp
from jax import lax
from jax.experimental import pallas as pl
from jax.experimental.pallas import tpu as pltpu
from jax.experimental.pallas import tpu_sc as plsc

NUM_CORES = 2
NUM_SUBCORES = 16
CHUNK = 128


def _edge_scatter(rows_tbl, srcp, dstp, n_pad, feat, const_rows):
    e_pad = dstp.shape[0]
    nw = NUM_CORES * NUM_SUBCORES
    cpt = e_pad // (nw * CHUNK)
    assert cpt % 2 == 0
    rpt = n_pad // NUM_SUBCORES
    mesh = plsc.VectorSubcoreMesh(core_axis_name="c", subcore_axis_name="s")
    zeros = jnp.zeros((rpt, feat), jnp.float32)
    src3 = srcp.reshape(nw, cpt, CHUNK)
    dst3 = dstp.reshape(nw, cpt, CHUNK)

    scratch = [
        pltpu.VMEM((cpt, CHUNK), jnp.int32),
        pltpu.VMEM((CHUNK, feat), jnp.float32),
        pltpu.VMEM((CHUNK, feat), jnp.float32),
        pltpu.VMEM_SHARED((n_pad, feat), jnp.float32),
        pltpu.SemaphoreType.DMA,
        pltpu.SemaphoreType.DMA,
    ]
    if not const_rows:
        scratch.insert(0, pltpu.VMEM((cpt, CHUNK), jnp.int32))

    @functools.partial(
        pl.kernel,
        mesh=mesh,
        out_type=jax.ShapeDtypeStruct((NUM_CORES * n_pad, feat), jnp.float32),
        scratch_types=scratch,
        compiler_params=pltpu.CompilerParams(use_tc_tiling_on_sc=False),
    )
    def k(rows_hbm, src_hbm, dst_hbm, zeros_hbm, out_hbm, *refs):
        if const_rows:
            srcv = None
            dstv, rows0, rows1, acc, sem0, sem1 = refs
        else:
            srcv, dstv, rows0, rows1, acc, sem0, sem1 = refs
        c = lax.axis_index("c")
        s = lax.axis_index("s")
        wid = c * NUM_SUBCORES + s
        pltpu.sync_copy(zeros_hbm, acc.at[pl.ds(s * rpt, rpt)])
        pltpu.sync_copy(dst_hbm.at[wid], dstv)
        if const_rows:
            pltpu.sync_copy(rows_hbm, rows0)
        else:
            pltpu.sync_copy(src_hbm.at[wid], srcv)
        plsc.subcore_barrier()

        if const_rows:
            def scat(i, sem):
                pltpu.async_copy(rows0, acc.at[dstv.at[i]], add=True, sem=sem)

            def wscat(sem):
                pltpu.make_async_copy(rows0, acc.at[dstv.at[0]], sem).wait()

            scat(0, sem0)
            scat(1, sem1)

            def body(kk, carry):
                wscat(sem0)
                scat(2 * kk, sem0)
                wscat(sem1)
                scat(2 * kk + 1, sem1)
                return carry

            lax.fori_loop(1, cpt // 2, body, 0)
            wscat(sem0)
            wscat(sem1)
        else:
            def gat(i, buf, sem):
                pltpu.async_copy(rows_hbm.at[srcv.at[i]], buf, sem)

            def wgat(buf, sem):
                pltpu.make_async_copy(rows_hbm.at[srcv.at[0]], buf, sem).wait()

            gat(0, rows0, sem0)

            def body(kk, carry):
                i0 = 2 * kk
                wgat(rows0, sem0)
                gat(i0 + 1, rows1, sem1)
                pltpu.sync_copy(rows0, acc.at[dstv.at[i0]], add=True)
                wgat(rows1, sem1)
                inext = jnp.minimum(i0 + 2, cpt - 1)
                gat(inext, rows0, sem0)
                pltpu.sync_copy(rows1, acc.at[dstv.at[i0 + 1]], add=True)
                return carry

            lax.fori_loop(0, cpt // 2, body, 0)
            wgat(rows0, sem0)

        plsc.subcore_barrier()
        pltpu.sync_copy(
            acc.at[pl.ds(s * rpt, rpt)],
            out_hbm.at[pl.ds(c * n_pad + s * rpt, rpt)],
        )

    return k(rows_tbl, src3, dst3, zeros)


def _dinv(d0_ref, d1_ref):
    deg = d0_ref[:, :1] + d1_ref[:, :1] + 1.0
    return lax.rsqrt(jnp.maximum(deg, 1.0))


def _enc1(x, W1, d0, d1):
    n, dfe = x.shape
    hid = W1.shape[1]
    blk = 2000

    def body(x_ref, w_ref, d0_ref, d1_ref, s_ref):
        dinv = _dinv(d0_ref, d1_ref)
        s_ref[...] = jnp.dot(x_ref[...], w_ref[...],
                             preferred_element_type=jnp.float32) * dinv

    return pl.pallas_call(
        body,
        grid=(n // blk,),
        in_specs=[
            pl.BlockSpec((blk, dfe), lambda i: (i, 0)),
            pl.BlockSpec((dfe, hid), lambda i: (0, 0)),
            pl.BlockSpec((blk, d0.shape[1]), lambda i: (i, 0)),
            pl.BlockSpec((blk, d1.shape[1]), lambda i: (i, 0)),
        ],
        out_specs=pl.BlockSpec((blk, hid), lambda i: (i, 0)),
        out_shape=jax.ShapeDtypeStruct((n, hid), jnp.float32),
    )(x, W1, d0, d1)


def _enc2(p0, p1, s, W2, d0, d1):
    n, hid = s.shape
    code = W2.shape[1]
    blk = 2000

    def body(p0_ref, p1_ref, s_ref, w_ref, d0_ref, d1_ref, t_ref):
        dinv = _dinv(d0_ref, d1_ref)
        h = jnp.maximum((p0_ref[...] + p1_ref[...] + s_ref[...]) * dinv, 0.0)
        t_ref[...] = jnp.dot(h, w_ref[...],
                             preferred_element_type=jnp.float32) * dinv

    rspec = lambda f: pl.BlockSpec((blk, f), lambda i: (i, 0))
    return pl.pallas_call(
        body,
        grid=(n // blk,),
        in_specs=[rspec(hid), rspec(hid), rspec(hid),
                  pl.BlockSpec((hid, code), lambda i: (0, 0)),
                  rspec(d0.shape[1]), rspec(d1.shape[1])],
        out_specs=rspec(code),
        out_shape=jax.ShapeDtypeStruct((n, code), jnp.float32),
    )(p0, p1, s, W2, d0, d1)


def _form_z(q0, q1, t, d0, d1):
    n, code = t.shape
    blk = 2000

    def body(q0_ref, q1_ref, t_ref, d0_ref, d1_ref, z_ref):
        dinv = _dinv(d0_ref, d1_ref)
        z_ref[...] = (q0_ref[...] + q1_ref[...] + t_ref[...]) * dinv

    rspec = lambda f: pl.BlockSpec((blk, f), lambda i: (i, 0))
    return pl.pallas_call(
        body,
        grid=(n // blk,),
        in_specs=[rspec(code), rspec(code), rspec(code),
                  rspec(d0.shape[1]), rspec(d1.shape[1])],
        out_specs=rspec(code),
        out_shape=jax.ShapeDtypeStruct((n, code), jnp.float32),
    )(q0, q1, t, d0, d1)


def _decode(z):
    n, code = z.shape
    rb, cb = 512, 1024
    gi = (n + rb - 1) // rb
    gj = (n + cb - 1) // cb

    def body(zi_ref, zj_ref, o_ref):
        g = lax.dot_general(zi_ref[...], zj_ref[...],
                            (((1,), (1,)), ((), ())),
                            preferred_element_type=jnp.float32)
        o_ref[...] = jax.nn.sigmoid(g)

    return pl.pallas_call(
        body,
        grid=(gi, gj),
        in_specs=[
            pl.BlockSpec((rb, code), lambda i, j: (i, 0)),
            pl.BlockSpec((cb, code), lambda i, j: (j, 0)),
        ],
        out_specs=pl.BlockSpec((rb, cb), lambda i, j: (i, j)),
        out_shape=jax.ShapeDtypeStruct((n, n), jnp.float32),
    )(z, z)


def kernel(x, edge_index, W1, W2):
    n = x.shape[0]
    e = edge_index.shape[1]

    epc = NUM_CORES * NUM_SUBCORES * CHUNK
    e_pad = ((e + epc - 1) // epc) * epc
    src = edge_index[0]
    dst = edge_index[1]
    srcp = jnp.concatenate([src, jnp.zeros((e_pad - e,), jnp.int32)])
    dstp = jnp.concatenate([dst, jnp.full((e_pad - e,), n, jnp.int32)])

    n_pad = ((n + 1 + NUM_SUBCORES * 8 - 1) // (NUM_SUBCORES * 8)) * (NUM_SUBCORES * 8)

    degf = 8
    ones = jnp.ones((CHUNK, degf), jnp.float32)
    degp = _edge_scatter(ones, srcp, dstp, n_pad, degf, const_rows=True)
    d0 = degp[:n]
    d1 = degp[n_pad:n_pad + n]

    s1 = _enc1(x, W1, d0, d1)
    pp = _edge_scatter(s1, srcp, dstp, n_pad, s1.shape[1], const_rows=False)
    t = _enc2(pp[:n], pp[n_pad:n_pad + n], s1, W2, d0, d1)
    qq = _edge_scatter(t, srcp, dstp, n_pad, t.shape[1], const_rows=False)
    z = _form_z(qq[:n], qq[n_pad:n_pad + n], t, d0, d1)
    return _decode(z)


# speedup vs baseline: 12.8041x; 1.2948x over previous
import functools

import jax
import jax.numpy as jnp
from jax import lax
from jax.experimental import pallas as pl
from jax.experimental.pallas import tpu as pltpu
from jax.experimental.pallas import tpu_sc as plsc

NUM_CORES = 2
NUM_SUBCORES = 16
CHUNK = 128


def _edge_scatter(rows_tbl, srcp, dstp, n_pad, feat, const_rows):
    e_pad = dstp.shape[0]
    nw = NUM_CORES * NUM_SUBCORES
    cpt = e_pad // (nw * CHUNK)
    assert cpt % 2 == 0
    rpt = n_pad // NUM_SUBCORES
    mesh = plsc.VectorSubcoreMesh(core_axis_name="c", subcore_axis_name="s")
    zeros = jnp.zeros((rpt, feat), jnp.float32)
    src3 = srcp.reshape(nw, cpt, CHUNK)
    dst3 = dstp.reshape(nw, cpt, CHUNK)

    scratch = [
        pltpu.VMEM((cpt, CHUNK), jnp.int32),
        pltpu.VMEM((CHUNK, feat), jnp.float32),
        pltpu.VMEM((CHUNK, feat), jnp.float32),
        pltpu.VMEM_SHARED((n_pad, feat), jnp.float32),
        pltpu.SemaphoreType.DMA,
        pltpu.SemaphoreType.DMA,
    ]
    if not const_rows:
        scratch.insert(0, pltpu.VMEM((cpt, CHUNK), jnp.int32))

    @functools.partial(
        pl.kernel,
        mesh=mesh,
        out_type=jax.ShapeDtypeStruct((NUM_CORES * n_pad, feat), jnp.float32),
        scratch_types=scratch,
        compiler_params=pltpu.CompilerParams(use_tc_tiling_on_sc=False),
    )
    def k(rows_hbm, src_hbm, dst_hbm, zeros_hbm, out_hbm, *refs):
        if const_rows:
            srcv = None
            dstv, rows0, rows1, acc, sem0, sem1 = refs
        else:
            srcv, dstv, rows0, rows1, acc, sem0, sem1 = refs
        c = lax.axis_index("c")
        s = lax.axis_index("s")
        wid = c * NUM_SUBCORES + s
        pltpu.sync_copy(zeros_hbm, acc.at[pl.ds(s * rpt, rpt)])
        pltpu.sync_copy(dst_hbm.at[wid], dstv)
        if const_rows:
            pltpu.sync_copy(rows_hbm, rows0)
        else:
            pltpu.sync_copy(src_hbm.at[wid], srcv)
        plsc.subcore_barrier()

        if const_rows:
            def scat(i, sem):
                pltpu.async_copy(rows0, acc.at[dstv.at[i]], add=True, sem=sem)

            def wscat(sem):
                pltpu.make_async_copy(rows0, acc.at[dstv.at[0]], sem).wait()

            scat(0, sem0)
            scat(1, sem1)

            def body(kk, carry):
                wscat(sem0)
                scat(2 * kk, sem0)
                wscat(sem1)
                scat(2 * kk + 1, sem1)
                return carry

            lax.fori_loop(1, cpt // 2, body, 0)
            wscat(sem0)
            wscat(sem1)
        else:
            def gat(i, buf, sem):
                pltpu.async_copy(rows_hbm.at[srcv.at[i]], buf, sem)

            def wgat(buf, sem):
                pltpu.make_async_copy(rows_hbm.at[srcv.at[0]], buf, sem).wait()

            gat(0, rows0, sem0)

            def body(kk, carry):
                i0 = 2 * kk
                wgat(rows0, sem0)
                gat(i0 + 1, rows1, sem1)
                pltpu.sync_copy(rows0, acc.at[dstv.at[i0]], add=True)
                wgat(rows1, sem1)
                inext = jnp.minimum(i0 + 2, cpt - 1)
                gat(inext, rows0, sem0)
                pltpu.sync_copy(rows1, acc.at[dstv.at[i0 + 1]], add=True)
                return carry

            lax.fori_loop(0, cpt // 2, body, 0)
            wgat(rows0, sem0)

        plsc.subcore_barrier()
        pltpu.sync_copy(
            acc.at[pl.ds(s * rpt, rpt)],
            out_hbm.at[pl.ds(c * n_pad + s * rpt, rpt)],
        )

    return k(rows_tbl, src3, dst3, zeros)


def _dinv(d0_ref, d1_ref):
    deg = d0_ref[:, :1] + d1_ref[:, :1] + 1.0
    return lax.rsqrt(jnp.maximum(deg, 1.0))


def _enc1(x, W1, d0, d1):
    n, dfe = x.shape
    hid = W1.shape[1]
    blk = 2000

    def body(x_ref, w_ref, d0_ref, d1_ref, s_ref):
        dinv = _dinv(d0_ref, d1_ref)
        s_ref[...] = jnp.dot(x_ref[...], w_ref[...],
                             preferred_element_type=jnp.float32) * dinv

    return pl.pallas_call(
        body,
        grid=(n // blk,),
        in_specs=[
            pl.BlockSpec((blk, dfe), lambda i: (i, 0)),
            pl.BlockSpec((dfe, hid), lambda i: (0, 0)),
            pl.BlockSpec((blk, d0.shape[1]), lambda i: (i, 0)),
            pl.BlockSpec((blk, d1.shape[1]), lambda i: (i, 0)),
        ],
        out_specs=pl.BlockSpec((blk, hid), lambda i: (i, 0)),
        out_shape=jax.ShapeDtypeStruct((n, hid), jnp.float32),
    )(x, W1, d0, d1)


def _enc2(p0, p1, s, W2, d0, d1):
    n, hid = s.shape
    code = W2.shape[1]
    blk = 2000

    def body(p0_ref, p1_ref, s_ref, w_ref, d0_ref, d1_ref, t_ref):
        dinv = _dinv(d0_ref, d1_ref)
        h = jnp.maximum((p0_ref[...] + p1_ref[...] + s_ref[...]) * dinv, 0.0)
        t_ref[...] = jnp.dot(h, w_ref[...],
                             preferred_element_type=jnp.float32) * dinv

    rspec = lambda f: pl.BlockSpec((blk, f), lambda i: (i, 0))
    return pl.pallas_call(
        body,
        grid=(n // blk,),
        in_specs=[rspec(hid), rspec(hid), rspec(hid),
                  pl.BlockSpec((hid, code), lambda i: (0, 0)),
                  rspec(d0.shape[1]), rspec(d1.shape[1])],
        out_specs=rspec(code),
        out_shape=jax.ShapeDtypeStruct((n, code), jnp.float32),
    )(p0, p1, s, W2, d0, d1)


def _form_z(q0, q1, t, d0, d1):
    n, code = t.shape
    blk = 2000

    def body(q0_ref, q1_ref, t_ref, d0_ref, d1_ref, z_ref):
        dinv = _dinv(d0_ref, d1_ref)
        z_ref[...] = (q0_ref[...] + q1_ref[...] + t_ref[...]) * dinv

    rspec = lambda f: pl.BlockSpec((blk, f), lambda i: (i, 0))
    return pl.pallas_call(
        body,
        grid=(n // blk,),
        in_specs=[rspec(code), rspec(code), rspec(code),
                  rspec(d0.shape[1]), rspec(d1.shape[1])],
        out_specs=rspec(code),
        out_shape=jax.ShapeDtypeStruct((n, code), jnp.float32),
    )(q0, q1, t, d0, d1)


def _decode(z):
    n, code = z.shape
    rb = 400

    def body(zi_ref, zj_ref, o_ref):
        g = lax.dot_general(zi_ref[...], zj_ref[...],
                            (((1,), (1,)), ((), ())),
                            preferred_element_type=jnp.float32)
        o_ref[...] = jax.nn.sigmoid(g)

    return pl.pallas_call(
        body,
        grid=(n // rb,),
        in_specs=[
            pl.BlockSpec((rb, code), lambda i: (i, 0)),
            pl.BlockSpec((n, code), lambda i: (0, 0)),
        ],
        out_specs=pl.BlockSpec((rb, n), lambda i: (i, 0)),
        out_shape=jax.ShapeDtypeStruct((n, n), jnp.float32),
    )(z, z)


def kernel(x, edge_index, W1, W2):
    n = x.shape[0]
    e = edge_index.shape[1]

    epc = NUM_CORES * NUM_SUBCORES * CHUNK
    e_pad = ((e + epc - 1) // epc) * epc
    src = edge_index[0]
    dst = edge_index[1]
    srcp = jnp.concatenate([src, jnp.zeros((e_pad - e,), jnp.int32)])
    dstp = jnp.concatenate([dst, jnp.full((e_pad - e,), n, jnp.int32)])

    n_pad = ((n + 1 + NUM_SUBCORES * 8 - 1) // (NUM_SUBCORES * 8)) * (NUM_SUBCORES * 8)

    degf = 8
    ones = jnp.ones((CHUNK, degf), jnp.float32)
    degp = _edge_scatter(ones, srcp, dstp, n_pad, degf, const_rows=True)
    d0 = degp[:n]
    d1 = degp[n_pad:n_pad + n]

    s1 = _enc1(x, W1, d0, d1)
    pp = _edge_scatter(s1, srcp, dstp, n_pad, s1.shape[1], const_rows=False)
    t = _enc2(pp[:n], pp[n_pad:n_pad + n], s1, W2, d0, d1)
    qq = _edge_scatter(t, srcp, dstp, n_pad, t.shape[1], const_rows=False)
    z = _form_z(qq[:n], qq[n_pad:n_pad + n], t, d0, d1)
    return _decode(z)


# speedup vs baseline: 13.4954x; 1.0540x over previous
import functools

import jax
import jax.numpy as jnp
from jax import lax
from jax.experimental import pallas as pl
from jax.experimental.pallas import tpu as pltpu
from jax.experimental.pallas import tpu_sc as plsc

NUM_CORES = 2
NUM_SUBCORES = 16
NW = NUM_CORES * NUM_SUBCORES
CHUNK = 128


def _edge_scatter(rows_tbl, ei4, n_pad, feat, const_rows):
    cpt = ei4.shape[2]
    assert cpt % 2 == 0
    rpt = n_pad // NUM_SUBCORES
    mesh = plsc.VectorSubcoreMesh(core_axis_name="c", subcore_axis_name="s")
    zeros = jnp.zeros((rpt, feat), jnp.float32)

    scratch = [
        pltpu.VMEM((cpt, CHUNK), jnp.int32),
        pltpu.VMEM((CHUNK, feat), jnp.float32),
        pltpu.VMEM((CHUNK, feat), jnp.float32),
        pltpu.VMEM_SHARED((n_pad, feat), jnp.float32),
        pltpu.SemaphoreType.DMA,
        pltpu.SemaphoreType.DMA,
    ]
    if not const_rows:
        scratch.insert(0, pltpu.VMEM((cpt, CHUNK), jnp.int32))

    @functools.partial(
        pl.kernel,
        mesh=mesh,
        out_type=jax.ShapeDtypeStruct((NUM_CORES, n_pad, feat), jnp.float32),
        scratch_types=scratch,
        compiler_params=pltpu.CompilerParams(use_tc_tiling_on_sc=False),
    )
    def k(rows_hbm, ei_hbm, zeros_hbm, out_hbm, *refs):
        if const_rows:
            srcv = None
            dstv, rows0, rows1, acc, sem0, sem1 = refs
        else:
            srcv, dstv, rows0, rows1, acc, sem0, sem1 = refs
        c = lax.axis_index("c")
        s = lax.axis_index("s")
        wid = c * NUM_SUBCORES + s
        pltpu.sync_copy(zeros_hbm, acc.at[pl.ds(s * rpt, rpt)])
        pltpu.sync_copy(ei_hbm.at[1, wid], dstv)
        if const_rows:
            pltpu.sync_copy(rows_hbm, rows0)
        else:
            pltpu.sync_copy(ei_hbm.at[0, wid], srcv)
        plsc.subcore_barrier()

        if const_rows:
            def scat(i, sem):
                pltpu.async_copy(rows0, acc.at[dstv.at[i]], sem, add=True)

            def wscat(sem):
                pltpu.make_async_copy(rows0, acc.at[dstv.at[0]], sem).wait()

            scat(0, sem0)
            scat(1, sem1)

            def body(kk, carry):
                wscat(sem0)
                scat(2 * kk, sem0)
                wscat(sem1)
                scat(2 * kk + 1, sem1)
                return carry

            lax.fori_loop(1, cpt // 2, body, 0)
            wscat(sem0)
            wscat(sem1)
        else:
            def gat(i, buf, sem):
                pltpu.async_copy(rows_hbm.at[srcv.at[i]], buf, sem)

            def wgat(buf, sem):
                pltpu.make_async_copy(rows_hbm.at[srcv.at[0]], buf, sem).wait()

            gat(0, rows0, sem0)

            def body(kk, carry):
                i0 = 2 * kk
                wgat(rows0, sem0)
                gat(i0 + 1, rows1, sem1)
                pltpu.sync_copy(rows0, acc.at[dstv.at[i0]], add=True)
                wgat(rows1, sem1)
                inext = jnp.minimum(i0 + 2, cpt - 1)
                gat(inext, rows0, sem0)
                pltpu.sync_copy(rows1, acc.at[dstv.at[i0 + 1]], add=True)
                return carry

            lax.fori_loop(0, cpt // 2, body, 0)
            wgat(rows0, sem0)

        plsc.subcore_barrier()
        pltpu.sync_copy(
            acc.at[pl.ds(s * rpt, rpt)],
            out_hbm.at[c, pl.ds(s * rpt, rpt)],
        )

    return k(rows_tbl, ei4, zeros)


def _dinv(dp_ref, n):
    dp = dp_ref[...]
    deg = dp[0, :n, :1] + dp[1, :n, :1] + 1.0
    return lax.rsqrt(jnp.maximum(deg, 1.0))


def _padrows(v, n_pad):
    n, f = v.shape
    return jnp.concatenate([v, jnp.zeros((n_pad - n, f), v.dtype)], axis=0)


def _full(shape):
    return pl.BlockSpec(shape, lambda i: (0,) * len(shape))


def _enc1(x, W1, degp, n_pad):
    n, dfe = x.shape
    hid = W1.shape[1]

    def body(x_ref, w_ref, dp_ref, s_ref):
        dinv = _dinv(dp_ref, n)
        s = jnp.dot(x_ref[...], w_ref[...],
                    preferred_element_type=jnp.float32) * dinv
        s_ref[...] = _padrows(s, n_pad)

    return pl.pallas_call(
        body,
        grid=(1,),
        in_specs=[_full((n, dfe)), _full((dfe, hid)), _full(degp.shape)],
        out_specs=_full((n_pad, hid)),
        out_shape=jax.ShapeDtypeStruct((n_pad, hid), jnp.float32),
    )(x, W1, degp)


def _enc2(pp, s, W2, degp, n):
    n_pad, hid = s.shape
    code = W2.shape[1]

    def body(pp_ref, s_ref, w_ref, dp_ref, t_ref):
        dinv = _dinv(dp_ref, n)
        p = pp_ref[...]
        h = jnp.maximum((p[0, :n] + p[1, :n] + s_ref[...][:n]) * dinv, 0.0)
        t = jnp.dot(h, w_ref[...],
                    preferred_element_type=jnp.float32) * dinv
        t_ref[...] = _padrows(t, n_pad)

    return pl.pallas_call(
        body,
        grid=(1,),
        in_specs=[_full(pp.shape), _full((n_pad, hid)), _full((hid, code)),
                  _full(degp.shape)],
        out_specs=_full((n_pad, code)),
        out_shape=jax.ShapeDtypeStruct((n_pad, code), jnp.float32),
    )(pp, s, W2, degp)


def _form_z(qq, t, degp, n):
    n_pad, code = t.shape

    def body(qq_ref, t_ref, dp_ref, z_ref):
        dinv = _dinv(dp_ref, n)
        q = qq_ref[...]
        z = (q[0, :n] + q[1, :n] + t_ref[...][:n]) * dinv
        z_ref[...] = _padrows(z, n_pad)

    return pl.pallas_call(
        body,
        grid=(1,),
        in_specs=[_full(qq.shape), _full((n_pad, code)), _full(degp.shape)],
        out_specs=_full((n_pad, code)),
        out_shape=jax.ShapeDtypeStruct((n_pad, code), jnp.float32),
    )(qq, t, degp)


def _decode(z, n):
    n_pad, code = z.shape
    rb = 400

    def body(zi_ref, zj_ref, o_ref):
        g = lax.dot_general(zi_ref[...], zj_ref[...],
                            (((1,), (1,)), ((), ())),
                            preferred_element_type=jnp.float32)
        o_ref[...] = jax.nn.sigmoid(g)

    return pl.pallas_call(
        body,
        grid=(n // rb,),
        in_specs=[
            pl.BlockSpec((rb, code), lambda i: (i, 0)),
            pl.BlockSpec((n, code), lambda i: (0, 0)),
        ],
        out_specs=pl.BlockSpec((rb, n), lambda i: (i, 0)),
        out_shape=jax.ShapeDtypeStruct((n, n), jnp.float32),
    )(z, z)


def kernel(x, edge_index, W1, W2):
    n = x.shape[0]
    e = edge_index.shape[1]

    epc = NW * CHUNK
    e_pad = ((e + epc - 1) // epc) * epc
    cpt = e_pad // epc
    ei4 = jnp.pad(edge_index, ((0, 0), (0, e_pad - e)),
                  constant_values=n).reshape(2, NW, cpt, CHUNK)

    n_pad = ((n + 1 + NUM_SUBCORES * 8 - 1) // (NUM_SUBCORES * 8)) * (NUM_SUBCORES * 8)

    degf = 8
    ones = jnp.ones((CHUNK, degf), jnp.float32)
    degp = _edge_scatter(ones, ei4, n_pad, degf, const_rows=True)

    s1 = _enc1(x, W1, degp, n_pad)
    pp = _edge_scatter(s1, ei4, n_pad, s1.shape[1], const_rows=False)
    t = _enc2(pp, s1, W2, degp, n)
    qq = _edge_scatter(t, ei4, n_pad, t.shape[1], const_rows=False)
    z = _form_z(qq, t, degp, n)
    return _decode(z, n)
